# baseline (device time: 113673 ns/iter reference)
import jax
import jax.numpy as jnp
from jax import lax
from jax.experimental import pallas as pl
from jax.experimental.pallas import tpu as pltpu

N_DEV = 16
N_TOK = 2048
D = 512
H = 1024
N_EXP = 64
E_LOCAL = N_EXP // N_DEV
CAP = 25
CAP_PAD = 32
BLOCK = E_LOCAL * CAP_PAD


def _moe_ring_kernel(xg, w_local):

    def body(xg_ref, w_ref, out_ref, mine_ref, copy_sem, send_sems, recv_sems):
        my = lax.axis_index("i")
        left = lax.rem(my - 1 + N_DEV, N_DEV)
        right = lax.rem(my + 1, N_DEV)

        barrier_sem = pltpu.get_barrier_semaphore()
        for nbr in (left, right):
            pl.semaphore_signal(
                barrier_sem, inc=1,
                device_id=(nbr,), device_id_type=pl.DeviceIdType.MESH,
            )
        pl.semaphore_wait(barrier_sem, 2)

        for k in range(E_LOCAL):
            blk = jnp.dot(
                xg_ref[k * CAP_PAD:(k + 1) * CAP_PAD, :],
                w_ref[k],
                preferred_element_type=jnp.float32,
            )
            mine_ref[k * CAP_PAD:(k + 1) * CAP_PAD, :] = blk.astype(mine_ref.dtype)

        local = pltpu.make_async_copy(mine_ref, out_ref.at[my], copy_sem)
        local.start()
        local.wait()

        for h in range(N_DEV - 1):
            send_origin = lax.rem(my - h + N_DEV, N_DEV)
            recv_origin = lax.rem(my - h - 1 + N_DEV, N_DEV)
            send = pltpu.make_async_remote_copy(
                src_ref=out_ref.at[send_origin],
                dst_ref=out_ref.at[send_origin],
                send_sem=send_sems.at[h],
                recv_sem=recv_sems.at[h],
                device_id=(right,),
                device_id_type=pl.DeviceIdType.MESH,
            )
            send.start()
            send.wait_send()
            recv = pltpu.make_async_remote_copy(
                src_ref=out_ref.at[recv_origin],
                dst_ref=out_ref.at[recv_origin],
                send_sem=send_sems.at[h],
                recv_sem=recv_sems.at[h],
                device_id=(left,),
                device_id_type=pl.DeviceIdType.MESH,
            )
            recv.wait_recv()

    return pl.pallas_call(
        body,
        out_shape=jax.ShapeDtypeStruct((N_DEV, BLOCK, H), jnp.bfloat16),
        in_specs=[
            pl.BlockSpec(memory_space=pltpu.VMEM),
            pl.BlockSpec(memory_space=pltpu.VMEM),
        ],
        out_specs=pl.BlockSpec(memory_space=pltpu.VMEM),
        scratch_shapes=[
            pltpu.VMEM((BLOCK, H), jnp.bfloat16),
            pltpu.SemaphoreType.DMA,
            pltpu.SemaphoreType.DMA((N_DEV - 1,)),
            pltpu.SemaphoreType.DMA((N_DEV - 1,)),
        ],
        compiler_params=pltpu.CompilerParams(collective_id=0),
    )(xg, w_local)


def kernel(x, router_W, route_idx, expert_W):
    pos = lax.axis_index("i")
    e = route_idx[:, 0].astype(jnp.int32)
    tok = jnp.arange(N_TOK, dtype=jnp.int32)

    onehot = (e[:, None] == jnp.arange(N_EXP, dtype=jnp.int32)[None, :]).astype(
        jnp.int32
    )
    cum = jnp.cumsum(onehot, axis=0)
    rank = jnp.take_along_axis(cum, e[:, None], axis=1)[:, 0] - 1

    local_e = pos * E_LOCAL + jnp.arange(E_LOCAL, dtype=jnp.int32)
    sel = (e[None, :] == local_e[:, None]) & (rank[None, :] < CAP)
    masked = jnp.where(sel, tok[None, :], N_TOK)
    gidx = jnp.sort(masked, axis=1)[:, :CAP_PAD]
    gidx = jnp.minimum(gidx, N_TOK - 1)

    xg = x[gidx.reshape(-1)].astype(jnp.bfloat16)
    wl = expert_W.astype(jnp.bfloat16)

    gathered = _moe_ring_kernel(xg, wl)
    gflat = gathered.reshape(N_DEV * BLOCK, H)

    my_tok = pos * (N_TOK // N_DEV) + jnp.arange(N_TOK // N_DEV, dtype=jnp.int32)
    e_t = jnp.take(e, my_tok)
    c_t = jnp.take(rank, my_tok)
    keep_t = c_t < CAP
    flat = (e_t // E_LOCAL) * BLOCK + (e_t % E_LOCAL) * CAP_PAD + c_t
    flat = jnp.where(keep_t, flat, 0)
    rows = jnp.take(gflat, flat, axis=0).astype(jnp.float32)
    return jnp.where(keep_t[:, None], rows, 0.0)


# device time: 86032 ns/iter; 1.3213x vs baseline; 1.3213x over previous
import jax
import jax.numpy as jnp
from jax import lax
from jax.experimental import pallas as pl
from jax.experimental.pallas import tpu as pltpu

N_DEV = 16
N_TOK = 2048
D = 512
H = 1024
N_EXP = 64
E_LOCAL = N_EXP // N_DEV
CAP = 25
CAP_PAD = 32
BLOCK = E_LOCAL * CAP_PAD
TB = 128
N_TB = N_TOK // TB


def _moe_kernel(x, route_idx, expert_W):
    def body(x_ref, e_ref, w_ref, out_ref, comm_ref, rank_ref, bs_ref,
             copy_sem, send_sems, recv_sems):
        my = lax.axis_index("i")
        left = lax.rem(my - 1 + N_DEV, N_DEV)
        right = lax.rem(my + 1, N_DEV)

        barrier_sem = pltpu.get_barrier_semaphore()
        for nbr in (left, right):
            pl.semaphore_signal(
                barrier_sem, inc=1,
                device_id=(nbr,), device_id_type=pl.DeviceIdType.MESH,
            )
        pl.semaphore_wait(barrier_sem, 2)

        eid = jnp.arange(N_EXP, dtype=jnp.int32)[None, :]
        tri = lax.broadcasted_iota(jnp.int32, (TB, TB), 1) <= \
            lax.broadcasted_iota(jnp.int32, (TB, TB), 0)
        l_incl = tri.astype(jnp.bfloat16)
        tri16 = lax.broadcasted_iota(jnp.int32, (N_TB, N_TB), 1) < \
            lax.broadcasted_iota(jnp.int32, (N_TB, N_TB), 0)
        l_excl16 = tri16.astype(jnp.bfloat16)

        for b in range(N_TB):
            e_b = e_ref[b * TB:(b + 1) * TB, :]
            oh = (e_b == eid).astype(jnp.float32)
            bs_ref[b:b + 1, :] = jnp.sum(oh, axis=0, keepdims=True)

        off = lax.dot_general(
            l_excl16, bs_ref[:, :].astype(jnp.bfloat16),
            (((1,), (0,)), ((), ())),
            preferred_element_type=jnp.float32,
        )

        for b in range(N_TB):
            e_b = e_ref[b * TB:(b + 1) * TB, :]
            oh32 = (e_b == eid).astype(jnp.float32)
            cum_b = lax.dot_general(
                l_incl, oh32.astype(jnp.bfloat16),
                (((1,), (0,)), ((), ())),
                preferred_element_type=jnp.float32,
            ) + off[b:b + 1, :]
            rank_b = jnp.sum(oh32 * cum_b, axis=1, keepdims=True) - 1.0
            rank_ref[b * TB:(b + 1) * TB, :] = rank_b

        e_all = e_ref[:, :]
        rank_all = rank_ref[:, :].astype(jnp.int32)
        keep = rank_all < CAP
        is_local = (e_all // E_LOCAL) == my
        slot = (e_all % E_LOCAL) * CAP_PAD + rank_all
        slot = jnp.where(keep & is_local, slot, -1)
        sel_t = (slot == lax.broadcasted_iota(jnp.int32, (N_TOK, BLOCK), 1))
        sel_t = sel_t.astype(jnp.bfloat16)
        xg = lax.dot_general(
            sel_t, x_ref[:, :].astype(jnp.bfloat16),
            (((0,), (0,)), ((), ())),
            preferred_element_type=jnp.float32,
        ).astype(jnp.bfloat16)

        for k in range(E_LOCAL):
            blk = jnp.dot(
                xg[k * CAP_PAD:(k + 1) * CAP_PAD, :],
                w_ref[k].astype(jnp.bfloat16),
                preferred_element_type=jnp.float32,
            )
            comm_ref[my, k * CAP_PAD:(k + 1) * CAP_PAD, :] = blk.astype(
                jnp.bfloat16
            )

        for h in range(N_DEV - 1):
            send_origin = lax.rem(my - h + N_DEV, N_DEV)
            recv_origin = lax.rem(my - h - 1 + N_DEV, N_DEV)
            send = pltpu.make_async_remote_copy(
                src_ref=comm_ref.at[send_origin],
                dst_ref=comm_ref.at[send_origin],
                send_sem=send_sems.at[h],
                recv_sem=recv_sems.at[h],
                device_id=(right,),
                device_id_type=pl.DeviceIdType.MESH,
            )
            send.start()
            send.wait_send()
            recv = pltpu.make_async_remote_copy(
                src_ref=comm_ref.at[recv_origin],
                dst_ref=comm_ref.at[recv_origin],
                send_sem=send_sems.at[h],
                recv_sem=recv_sems.at[h],
                device_id=(left,),
                device_id_type=pl.DeviceIdType.MESH,
            )
            recv.wait_recv()

        e_my = e_ref[pl.ds(my * (N_TOK // N_DEV), N_TOK // N_DEV), :]
        r_my = rank_ref[pl.ds(my * (N_TOK // N_DEV), N_TOK // N_DEV), :]
        r_my = r_my.astype(jnp.int32)
        flat = (e_my // E_LOCAL) * BLOCK + (e_my % E_LOCAL) * CAP_PAD + r_my
        flat = jnp.where(r_my < CAP, flat, -1)
        sel_o = (flat == lax.broadcasted_iota(
            jnp.int32, (N_TOK // N_DEV, N_DEV * BLOCK), 1))
        sel_o = sel_o.astype(jnp.bfloat16)
        g = comm_ref[...].reshape(N_DEV * BLOCK, H)
        out_ref[:, :] = lax.dot_general(
            sel_o, g, (((1,), (0,)), ((), ())),
            preferred_element_type=jnp.float32,
        )

    return pl.pallas_call(
        body,
        out_shape=jax.ShapeDtypeStruct((N_TOK // N_DEV, H), jnp.float32),
        in_specs=[
            pl.BlockSpec(memory_space=pltpu.VMEM),
            pl.BlockSpec(memory_space=pltpu.VMEM),
            pl.BlockSpec(memory_space=pltpu.VMEM),
        ],
        out_specs=pl.BlockSpec(memory_space=pltpu.VMEM),
        scratch_shapes=[
            pltpu.VMEM((N_DEV, BLOCK, H), jnp.bfloat16),
            pltpu.VMEM((N_TOK, 1), jnp.float32),
            pltpu.VMEM((N_TB, N_EXP), jnp.float32),
            pltpu.SemaphoreType.DMA,
            pltpu.SemaphoreType.DMA((N_DEV - 1,)),
            pltpu.SemaphoreType.DMA((N_DEV - 1,)),
        ],
        compiler_params=pltpu.CompilerParams(collective_id=0),
    )(x, route_idx, expert_W)


def kernel(x, router_W, route_idx, expert_W):
    return _moe_kernel(x, route_idx.astype(jnp.int32), expert_W)


# device time: 54508 ns/iter; 2.0854x vs baseline; 1.5783x over previous
import jax
import jax.numpy as jnp
from jax import lax
from jax.experimental import pallas as pl
from jax.experimental.pallas import tpu as pltpu

N_DEV = 16
N_TOK = 2048
D = 512
H = 1024
N_EXP = 64
E_LOCAL = N_EXP // N_DEV
CAP = 25
CAP_PAD = 32
BLOCK = E_LOCAL * CAP_PAD
TB = 128
N_TB = N_TOK // TB


def _moe_kernel(x, route_idx, expert_W):
    def body(x_ref, e_ref, w_ref, out_ref, comm_ref, rank_ref, bs_ref,
             send_cw, recv_cw, send_ccw, recv_ccw):
        my = lax.axis_index("i")
        left = lax.rem(my - 1 + N_DEV, N_DEV)
        right = lax.rem(my + 1, N_DEV)

        barrier_sem = pltpu.get_barrier_semaphore()
        for nbr in (left, right):
            pl.semaphore_signal(
                barrier_sem, inc=1,
                device_id=(nbr,), device_id_type=pl.DeviceIdType.MESH,
            )
        pl.semaphore_wait(barrier_sem, 2)

        eid = jnp.arange(N_EXP, dtype=jnp.int32)[None, :]
        tri = lax.broadcasted_iota(jnp.int32, (TB, TB), 1) <= \
            lax.broadcasted_iota(jnp.int32, (TB, TB), 0)
        l_incl = tri.astype(jnp.bfloat16)
        tri16 = lax.broadcasted_iota(jnp.int32, (N_TB, N_TB), 1) < \
            lax.broadcasted_iota(jnp.int32, (N_TB, N_TB), 0)
        l_excl16 = tri16.astype(jnp.bfloat16)

        for b in range(N_TB):
            e_b = e_ref[b * TB:(b + 1) * TB, :]
            oh = (e_b == eid).astype(jnp.float32)
            bs_ref[b:b + 1, :] = jnp.sum(oh, axis=0, keepdims=True)

        off = lax.dot_general(
            l_excl16, bs_ref[:, :].astype(jnp.bfloat16),
            (((1,), (0,)), ((), ())),
            preferred_element_type=jnp.float32,
        )

        for b in range(N_TB):
            e_b = e_ref[b * TB:(b + 1) * TB, :]
            oh32 = (e_b == eid).astype(jnp.float32)
            cum_b = lax.dot_general(
                l_incl, oh32.astype(jnp.bfloat16),
                (((1,), (0,)), ((), ())),
                preferred_element_type=jnp.float32,
            ) + off[b:b + 1, :]
            rank_b = jnp.sum(oh32 * cum_b, axis=1, keepdims=True) - 1.0
            rank_ref[b * TB:(b + 1) * TB, :] = rank_b

        e_all = e_ref[:, :]
        rank_all = rank_ref[:, :].astype(jnp.int32)
        keep = rank_all < CAP
        is_local = (e_all // E_LOCAL) == my
        slot = (e_all % E_LOCAL) * CAP_PAD + rank_all
        slot = jnp.where(keep & is_local, slot, -1)
        sel_t = (slot == lax.broadcasted_iota(jnp.int32, (N_TOK, BLOCK), 1))
        sel_t = sel_t.astype(jnp.bfloat16)
        xg = lax.dot_general(
            sel_t, x_ref[:, :].astype(jnp.bfloat16),
            (((0,), (0,)), ((), ())),
            preferred_element_type=jnp.float32,
        ).astype(jnp.bfloat16)

        for k in range(E_LOCAL):
            blk = jnp.dot(
                xg[k * CAP_PAD:(k + 1) * CAP_PAD, :],
                w_ref[k].astype(jnp.bfloat16),
                preferred_element_type=jnp.float32,
            )
            comm_ref[my, k * CAP_PAD:(k + 1) * CAP_PAD, :] = blk.astype(
                jnp.bfloat16
            )

        CW, CCW = N_DEV // 2, N_DEV // 2 - 1

        def cw_send(h):
            origin = lax.rem(my - h + N_DEV, N_DEV)
            d = pltpu.make_async_remote_copy(
                src_ref=comm_ref.at[origin],
                dst_ref=comm_ref.at[origin],
                send_sem=send_cw.at[h],
                recv_sem=recv_cw.at[h],
                device_id=(right,),
                device_id_type=pl.DeviceIdType.MESH,
            )
            d.start()
            return d

        def ccw_send(h):
            origin = lax.rem(my + h, N_DEV)
            d = pltpu.make_async_remote_copy(
                src_ref=comm_ref.at[origin],
                dst_ref=comm_ref.at[origin],
                send_sem=send_ccw.at[h],
                recv_sem=recv_ccw.at[h],
                device_id=(left,),
                device_id_type=pl.DeviceIdType.MESH,
            )
            d.start()
            return d

        def cw_recv_wait(h):
            origin = lax.rem(my - h - 1 + N_DEV, N_DEV)
            pltpu.make_async_remote_copy(
                src_ref=comm_ref.at[origin],
                dst_ref=comm_ref.at[origin],
                send_sem=send_cw.at[h],
                recv_sem=recv_cw.at[h],
                device_id=(left,),
                device_id_type=pl.DeviceIdType.MESH,
            ).wait_recv()

        def ccw_recv_wait(h):
            origin = lax.rem(my + h + 1, N_DEV)
            pltpu.make_async_remote_copy(
                src_ref=comm_ref.at[origin],
                dst_ref=comm_ref.at[origin],
                send_sem=send_ccw.at[h],
                recv_sem=recv_ccw.at[h],
                device_id=(right,),
                device_id_type=pl.DeviceIdType.MESH,
            ).wait_recv()

        in_flight = [cw_send(0), ccw_send(0)]
        for h in range(CW):
            cw_recv_wait(h)
            if h + 1 < CW:
                in_flight.append(cw_send(h + 1))
            if h < CCW:
                ccw_recv_wait(h)
                if h + 1 < CCW:
                    in_flight.append(ccw_send(h + 1))

        e_my = e_ref[pl.ds(my * (N_TOK // N_DEV), N_TOK // N_DEV), :]
        r_my = rank_ref[pl.ds(my * (N_TOK // N_DEV), N_TOK // N_DEV), :]
        r_my = r_my.astype(jnp.int32)
        flat = (e_my // E_LOCAL) * BLOCK + (e_my % E_LOCAL) * CAP_PAD + r_my
        flat = jnp.where(r_my < CAP, flat, -1)
        sel_o = (flat == lax.broadcasted_iota(
            jnp.int32, (N_TOK // N_DEV, N_DEV * BLOCK), 1))
        sel_o = sel_o.astype(jnp.bfloat16)
        g = comm_ref[...].reshape(N_DEV * BLOCK, H)
        out_ref[:, :] = lax.dot_general(
            sel_o, g, (((1,), (0,)), ((), ())),
            preferred_element_type=jnp.float32,
        )

        for d in in_flight:
            d.wait_send()

    return pl.pallas_call(
        body,
        out_shape=jax.ShapeDtypeStruct((N_TOK // N_DEV, H), jnp.float32),
        in_specs=[
            pl.BlockSpec(memory_space=pltpu.VMEM),
            pl.BlockSpec(memory_space=pltpu.VMEM),
            pl.BlockSpec(memory_space=pltpu.VMEM),
        ],
        out_specs=pl.BlockSpec(memory_space=pltpu.VMEM),
        scratch_shapes=[
            pltpu.VMEM((N_DEV, BLOCK, H), jnp.bfloat16),
            pltpu.VMEM((N_TOK, 1), jnp.float32),
            pltpu.VMEM((N_TB, N_EXP), jnp.float32),
            pltpu.SemaphoreType.DMA((N_DEV // 2,)),
            pltpu.SemaphoreType.DMA((N_DEV // 2,)),
            pltpu.SemaphoreType.DMA((N_DEV // 2 - 1,)),
            pltpu.SemaphoreType.DMA((N_DEV // 2 - 1,)),
        ],
        compiler_params=pltpu.CompilerParams(collective_id=0),
    )(x, route_idx, expert_W)


def kernel(x, router_W, route_idx, expert_W):
    return _moe_kernel(x, route_idx.astype(jnp.int32), expert_W)


# device time: 47934 ns/iter; 2.3714x vs baseline; 1.1371x over previous
import jax
import jax.numpy as jnp
from jax import lax
from jax.experimental import pallas as pl
from jax.experimental.pallas import tpu as pltpu

N_DEV = 16
N_TOK = 2048
D = 512
H = 1024
N_EXP = 64
E_LOCAL = N_EXP // N_DEV
CAP = 25
CAP_PAD = 32
BLOCK = E_LOCAL * CAP_PAD
TB = 128
N_TB = N_TOK // TB
SUB = 4


def _moe_kernel(x, route_idx, expert_W):
    def body(x_ref, e_ref, w_ref, out_ref, comm_ref, rank_ref, bs_ref,
             send_cw, recv_cw, send_ccw, recv_ccw):
        my = lax.axis_index("i")
        left = lax.rem(my - 1 + N_DEV, N_DEV)
        right = lax.rem(my + 1, N_DEV)

        barrier_sem = pltpu.get_barrier_semaphore()
        for nbr in (left, right):
            pl.semaphore_signal(
                barrier_sem, inc=1,
                device_id=(nbr,), device_id_type=pl.DeviceIdType.MESH,
            )
        pl.semaphore_wait(barrier_sem, 2)

        eid = jnp.arange(N_EXP, dtype=jnp.int32)[None, :]
        tri = lax.broadcasted_iota(jnp.int32, (TB, TB), 1) <= \
            lax.broadcasted_iota(jnp.int32, (TB, TB), 0)
        l_incl = tri.astype(jnp.bfloat16)
        tri16 = lax.broadcasted_iota(jnp.int32, (N_TB, N_TB), 1) < \
            lax.broadcasted_iota(jnp.int32, (N_TB, N_TB), 0)
        l_excl16 = tri16.astype(jnp.bfloat16)

        for b in range(N_TB):
            e_b = e_ref[b * TB:(b + 1) * TB, :]
            oh = (e_b == eid).astype(jnp.float32)
            bs_ref[b:b + 1, :] = jnp.sum(oh, axis=0, keepdims=True)

        off = lax.dot_general(
            l_excl16, bs_ref[:, :].astype(jnp.bfloat16),
            (((1,), (0,)), ((), ())),
            preferred_element_type=jnp.float32,
        )

        for b in range(N_TB):
            e_b = e_ref[b * TB:(b + 1) * TB, :]
            oh32 = (e_b == eid).astype(jnp.float32)
            cum_b = lax.dot_general(
                l_incl, oh32.astype(jnp.bfloat16),
                (((1,), (0,)), ((), ())),
                preferred_element_type=jnp.float32,
            ) + off[b:b + 1, :]
            rank_b = jnp.sum(oh32 * cum_b, axis=1, keepdims=True) - 1.0
            rank_ref[b * TB:(b + 1) * TB, :] = rank_b

        e_all = e_ref[:, :]
        rank_all = rank_ref[:, :].astype(jnp.int32)
        keep = rank_all < CAP
        is_local = (e_all // E_LOCAL) == my
        slot = (e_all % E_LOCAL) * CAP_PAD + rank_all
        slot = jnp.where(keep & is_local, slot, -1)
        sel_t = (slot == lax.broadcasted_iota(jnp.int32, (N_TOK, BLOCK), 1))
        sel_t = sel_t.astype(jnp.bfloat16)
        xg = lax.dot_general(
            sel_t, x_ref[:, :].astype(jnp.bfloat16),
            (((0,), (0,)), ((), ())),
            preferred_element_type=jnp.float32,
        ).astype(jnp.bfloat16)

        for k in range(E_LOCAL):
            blk = jnp.dot(
                xg[k * CAP_PAD:(k + 1) * CAP_PAD, :],
                w_ref[k].astype(jnp.bfloat16),
                preferred_element_type=jnp.float32,
            )
            comm_ref[my, k * CAP_PAD:(k + 1) * CAP_PAD, :] = blk.astype(
                jnp.bfloat16
            )

        CW, CCW = N_DEV // 2, N_DEV // 2 - 1
        SROWS = BLOCK // SUB

        def _mk(h, s, origin, nbr, ssem, rsem):
            return pltpu.make_async_remote_copy(
                src_ref=comm_ref.at[origin, pl.ds(s * SROWS, SROWS)],
                dst_ref=comm_ref.at[origin, pl.ds(s * SROWS, SROWS)],
                send_sem=ssem.at[h, s],
                recv_sem=rsem.at[h, s],
                device_id=(nbr,),
                device_id_type=pl.DeviceIdType.MESH,
            )

        def cw_send(h, s):
            d = _mk(h, s, lax.rem(my - h + N_DEV, N_DEV), right,
                    send_cw, recv_cw)
            d.start()
            return d

        def ccw_send(h, s):
            d = _mk(h, s, lax.rem(my + h, N_DEV), left, send_ccw, recv_ccw)
            d.start()
            return d

        def cw_recv_wait(h, s):
            _mk(h, s, lax.rem(my - h - 1 + N_DEV, N_DEV), left,
                send_cw, recv_cw).wait_recv()

        def ccw_recv_wait(h, s):
            _mk(h, s, lax.rem(my + h + 1, N_DEV), right,
                send_ccw, recv_ccw).wait_recv()

        in_flight = []
        for s in range(SUB):
            in_flight.append(cw_send(0, s))
            in_flight.append(ccw_send(0, s))
        for h in range(CW):
            for s in range(SUB):
                cw_recv_wait(h, s)
                if h + 1 < CW:
                    in_flight.append(cw_send(h + 1, s))
                if h < CCW:
                    ccw_recv_wait(h, s)
                    if h + 1 < CCW:
                        in_flight.append(ccw_send(h + 1, s))

        e_my = e_ref[pl.ds(my * (N_TOK // N_DEV), N_TOK // N_DEV), :]
        r_my = rank_ref[pl.ds(my * (N_TOK // N_DEV), N_TOK // N_DEV), :]
        r_my = r_my.astype(jnp.int32)
        flat = (e_my // E_LOCAL) * BLOCK + (e_my % E_LOCAL) * CAP_PAD + r_my
        flat = jnp.where(r_my < CAP, flat, -1)
        sel_o = (flat == lax.broadcasted_iota(
            jnp.int32, (N_TOK // N_DEV, N_DEV * BLOCK), 1))
        sel_o = sel_o.astype(jnp.bfloat16)
        g = comm_ref[...].reshape(N_DEV * BLOCK, H)
        out_ref[:, :] = lax.dot_general(
            sel_o, g, (((1,), (0,)), ((), ())),
            preferred_element_type=jnp.float32,
        )

        for d in in_flight:
            d.wait_send()

    return pl.pallas_call(
        body,
        out_shape=jax.ShapeDtypeStruct((N_TOK // N_DEV, H), jnp.float32),
        in_specs=[
            pl.BlockSpec(memory_space=pltpu.VMEM),
            pl.BlockSpec(memory_space=pltpu.VMEM),
            pl.BlockSpec(memory_space=pltpu.VMEM),
        ],
        out_specs=pl.BlockSpec(memory_space=pltpu.VMEM),
        scratch_shapes=[
            pltpu.VMEM((N_DEV, BLOCK, H), jnp.bfloat16),
            pltpu.VMEM((N_TOK, 1), jnp.float32),
            pltpu.VMEM((N_TB, N_EXP), jnp.float32),
            pltpu.SemaphoreType.DMA((N_DEV // 2, SUB)),
            pltpu.SemaphoreType.DMA((N_DEV // 2, SUB)),
            pltpu.SemaphoreType.DMA((N_DEV // 2 - 1, SUB)),
            pltpu.SemaphoreType.DMA((N_DEV // 2 - 1, SUB)),
        ],
        compiler_params=pltpu.CompilerParams(collective_id=0),
    )(x, route_idx, expert_W)


def kernel(x, router_W, route_idx, expert_W):
    return _moe_kernel(x, route_idx.astype(jnp.int32), expert_W)


# device time: 13253 ns/iter; 8.5772x vs baseline; 3.6168x over previous
import jax
import jax.numpy as jnp
from jax import lax
from jax.experimental import pallas as pl
from jax.experimental.pallas import tpu as pltpu

N_DEV = 16
N_TOK = 2048
D = 512
H = 1024
N_EXP = 64
E_LOCAL = N_EXP // N_DEV
CAP = 25
CAP_PAD = 32
BLOCK = E_LOCAL * CAP_PAD
TB = 128
N_TB = N_TOK // TB
SUB = 4
_RING = True


def _moe_kernel(x, route_idx, expert_W):
    def body(x_ref, e_ref, w_ref, out_ref, comm_ref, rank_ref, bs_ref,
             intra_ref, sel_ref, send_cw, recv_cw, send_ccw, recv_ccw):
        my = lax.axis_index("i")
        left = lax.rem(my - 1 + N_DEV, N_DEV)
        right = lax.rem(my + 1, N_DEV)

        barrier_sem = pltpu.get_barrier_semaphore()
        for nbr in (left, right):
            pl.semaphore_signal(
                barrier_sem, inc=1,
                device_id=(nbr,), device_id_type=pl.DeviceIdType.MESH,
            )
        pl.semaphore_wait(barrier_sem, 2)

        eid = jnp.arange(N_EXP, dtype=jnp.int32)[None, :]
        tri = lax.broadcasted_iota(jnp.int32, (TB, TB), 1) <= \
            lax.broadcasted_iota(jnp.int32, (TB, TB), 0)
        l_incl = tri.astype(jnp.bfloat16)
        tri16 = lax.broadcasted_iota(jnp.int32, (N_TB, N_TB), 1) < \
            lax.broadcasted_iota(jnp.int32, (N_TB, N_TB), 0)
        l_excl16 = tri16.astype(jnp.bfloat16)

        oh_bf = (e_ref[:, :] == eid).astype(jnp.bfloat16)

        for b in range(N_TB):
            intra_b = lax.dot_general(
                l_incl, oh_bf[b * TB:(b + 1) * TB, :],
                (((1,), (0,)), ((), ())),
                preferred_element_type=jnp.float32,
            )
            intra_ref[b * TB:(b + 1) * TB, :] = intra_b
            bs_ref[b:b + 1, :] = intra_b[TB - 1:TB, :]

        off = lax.dot_general(
            l_excl16, bs_ref[:, :].astype(jnp.bfloat16),
            (((1,), (0,)), ((), ())),
            preferred_element_type=jnp.float32,
        )
        rep = (lax.broadcasted_iota(jnp.int32, (N_TOK, N_TB), 0) // TB ==
               lax.broadcasted_iota(jnp.int32, (N_TOK, N_TB), 1))
        off_rep = lax.dot_general(
            rep.astype(jnp.bfloat16), off.astype(jnp.bfloat16),
            (((1,), (0,)), ((), ())),
            preferred_element_type=jnp.float32,
        )
        oh32 = oh_bf.astype(jnp.float32)
        rank_ref[:, :] = jnp.sum(
            oh32 * (intra_ref[:, :] + off_rep), axis=1, keepdims=True
        ) - 1.0

        e_all = e_ref[:, :]
        rank_all = rank_ref[:, :].astype(jnp.int32)
        keep = rank_all < CAP
        is_local = (e_all // E_LOCAL) == my
        slot = (e_all % E_LOCAL) * CAP_PAD + rank_all
        slot = jnp.where(keep & is_local, slot, -1)
        sel_t = (slot == lax.broadcasted_iota(jnp.int32, (N_TOK, BLOCK), 1))
        sel_t = sel_t.astype(jnp.bfloat16)
        xg = lax.dot_general(
            sel_t, x_ref[:, :].astype(jnp.bfloat16),
            (((0,), (0,)), ((), ())),
            preferred_element_type=jnp.float32,
        ).astype(jnp.bfloat16)

        for k in range(E_LOCAL):
            blk = jnp.dot(
                xg[k * CAP_PAD:(k + 1) * CAP_PAD, :],
                w_ref[k].astype(jnp.bfloat16),
                preferred_element_type=jnp.float32,
            )
            comm_ref[my, k * CAP_PAD:(k + 1) * CAP_PAD, :] = blk.astype(
                jnp.bfloat16
            )

        e_my = e_ref[pl.ds(my * (N_TOK // N_DEV), N_TOK // N_DEV), :]
        r_my = rank_ref[pl.ds(my * (N_TOK // N_DEV), N_TOK // N_DEV), :]
        r_my = r_my.astype(jnp.int32)
        flat = (e_my // E_LOCAL) * BLOCK + (e_my % E_LOCAL) * CAP_PAD + r_my
        flat = jnp.where(r_my < CAP, flat, -1)
        sel_ref[:, :] = (flat == lax.broadcasted_iota(
            jnp.int32, (N_TOK // N_DEV, N_DEV * BLOCK), 1)).astype(jnp.bfloat16)

        def acc_origin(acc, origin):
            return acc + lax.dot_general(
                sel_ref[:, pl.ds(origin * BLOCK, BLOCK)],
                comm_ref[origin],
                (((1,), (0,)), ((), ())),
                preferred_element_type=jnp.float32,
            )

        CW, CCW = N_DEV // 2, N_DEV // 2 - 1
        SROWS = BLOCK // SUB

        def _mk(h, s, origin, nbr, ssem, rsem):
            return pltpu.make_async_remote_copy(
                src_ref=comm_ref.at[origin, pl.ds(s * SROWS, SROWS)],
                dst_ref=comm_ref.at[origin, pl.ds(s * SROWS, SROWS)],
                send_sem=ssem.at[h, s],
                recv_sem=rsem.at[h, s],
                device_id=(nbr,),
                device_id_type=pl.DeviceIdType.MESH,
            )

        def cw_send(h, s):
            d = _mk(h, s, lax.rem(my - h + N_DEV, N_DEV), right,
                    send_cw, recv_cw)
            d.start()
            return d

        def ccw_send(h, s):
            d = _mk(h, s, lax.rem(my + h, N_DEV), left, send_ccw, recv_ccw)
            d.start()
            return d

        def cw_recv_wait(h, s):
            _mk(h, s, lax.rem(my - h - 1 + N_DEV, N_DEV), left,
                send_cw, recv_cw).wait_recv()

        def ccw_recv_wait(h, s):
            _mk(h, s, lax.rem(my + h + 1, N_DEV), right,
                send_ccw, recv_ccw).wait_recv()

        in_flight = []
        acc = jnp.zeros((N_TOK // N_DEV, H), dtype=jnp.float32)
        acc = acc_origin(acc, my)
        if _RING:
            for s in range(SUB):
                in_flight.append(cw_send(0, s))
                in_flight.append(ccw_send(0, s))
            for h in range(CW):
                for s in range(SUB):
                    cw_recv_wait(h, s)
                    if h + 1 < CW:
                        in_flight.append(cw_send(h + 1, s))
                    if h < CCW:
                        ccw_recv_wait(h, s)
                        if h + 1 < CCW:
                            in_flight.append(ccw_send(h + 1, s))
                acc = acc_origin(acc, lax.rem(my - h - 1 + N_DEV, N_DEV))
                if h < CCW:
                    acc = acc_origin(acc, lax.rem(my + h + 1, N_DEV))

        out_ref[:, :] = acc

        for d in in_flight:
            d.wait_send()

    return pl.pallas_call(
        body,
        out_shape=jax.ShapeDtypeStruct((N_TOK // N_DEV, H), jnp.float32),
        in_specs=[
            pl.BlockSpec(memory_space=pltpu.VMEM),
            pl.BlockSpec(memory_space=pltpu.VMEM),
            pl.BlockSpec(memory_space=pltpu.VMEM),
        ],
        out_specs=pl.BlockSpec(memory_space=pltpu.VMEM),
        scratch_shapes=[
            pltpu.VMEM((N_DEV, BLOCK, H), jnp.bfloat16),
            pltpu.VMEM((N_TOK, 1), jnp.float32),
            pltpu.VMEM((N_TB, N_EXP), jnp.float32),
            pltpu.VMEM((N_TOK, N_EXP), jnp.float32),
            pltpu.VMEM((N_TOK // N_DEV, N_DEV * BLOCK), jnp.bfloat16),
            pltpu.SemaphoreType.DMA((N_DEV // 2, SUB)),
            pltpu.SemaphoreType.DMA((N_DEV // 2, SUB)),
            pltpu.SemaphoreType.DMA((N_DEV // 2 - 1, SUB)),
            pltpu.SemaphoreType.DMA((N_DEV // 2 - 1, SUB)),
        ],
        compiler_params=pltpu.CompilerParams(collective_id=0),
    )(x, route_idx, expert_W)


def kernel(x, router_W, route_idx, expert_W):
    return _moe_kernel(x, route_idx.astype(jnp.int32), expert_W)
